# baseline (device time: 188485 ns/iter reference)
import jax
import jax.numpy as jnp
from jax import lax
from jax.experimental import pallas as pl
from jax.experimental.pallas import tpu as pltpu

N_DEV = 16
CW_HOPS = 8
CCW_HOPS = 7


def kernel(A, B):
    m_per, k = A.shape
    _, n = B.shape
    out_m = N_DEV * m_per

    def body(a_ref, b_ref, out_ref,
             cw_ref, ccw_ref, cmp_ref,
             cw_send, cw_recv, ccw_send, ccw_recv,
             res_ref, st0_sems, st1_sems, b_bf,
             cw_credit, ccw_credit):
        my = lax.axis_index("i")
        left = (my - 1) % N_DEV
        right = (my + 1) % N_DEV

        barrier_sem = pltpu.get_barrier_semaphore()
        for nbr in (left, right):
            pl.semaphore_signal(
                barrier_sem, inc=1,
                device_id=(nbr,), device_id_type=pl.DeviceIdType.MESH,
            )
        pl.semaphore_wait(barrier_sem, 2)

        a_bf = a_ref[:, :].astype(jnp.bfloat16)
        b_bf[:, :] = b_ref[:, :].astype(jnp.bfloat16)

        cw_ref[0, :, :] = a_bf
        ccw_ref[0, :, :] = a_bf

        pl.semaphore_signal(
            cw_credit, inc=1,
            device_id=(left,), device_id_type=pl.DeviceIdType.MESH,
        )
        pl.semaphore_signal(
            ccw_credit, inc=1,
            device_id=(right,), device_id_type=pl.DeviceIdType.MESH,
        )

        stores = {}

        def drain(key):
            if key in stores:
                stores[key].wait()
                del stores[key]

        def store_half(res_slot, half, origin):
            sems = st0_sems if half == 0 else st1_sems
            cp = pltpu.make_async_copy(
                res_ref.at[res_slot, pl.ds(half * m_per, m_per), :],
                out_ref.at[pl.ds(origin * m_per, m_per), :],
                sems.at[res_slot],
            )
            cp.start()
            stores[(res_slot, half)] = cp

        for h in range(CW_HOPS):
            ss = h % 2
            rs = (h + 1) % 2
            res_slot = h % 2

            pl.semaphore_wait(cw_credit, 1)
            cw_rdma = pltpu.make_async_remote_copy(
                src_ref=cw_ref.at[ss],
                dst_ref=cw_ref.at[rs],
                send_sem=cw_send.at[ss],
                recv_sem=cw_recv.at[rs],
                device_id=(right,),
                device_id_type=pl.DeviceIdType.MESH,
            )
            cw_rdma.start()
            if h < CCW_HOPS:
                pl.semaphore_wait(ccw_credit, 1)
                ccw_rdma = pltpu.make_async_remote_copy(
                    src_ref=ccw_ref.at[ss],
                    dst_ref=ccw_ref.at[rs],
                    send_sem=ccw_send.at[ss],
                    recv_sem=ccw_recv.at[rs],
                    device_id=(left,),
                    device_id_type=pl.DeviceIdType.MESH,
                )
                ccw_rdma.start()

            drain((res_slot, 0))
            drain((res_slot, 1))
            if h == 0:
                res_ref[0, pl.ds(0, m_per), :] = jnp.dot(
                    a_bf, b_bf[:, :], preferred_element_type=jnp.float32
                )
                store_half(0, 0, my)
            else:
                res_ref[res_slot, :, :] = jnp.dot(
                    cmp_ref[ss].reshape(2 * m_per, k), b_bf[:, :],
                    preferred_element_type=jnp.float32,
                )
                store_half(res_slot, 0, (my - h) % N_DEV)
                store_half(res_slot, 1, (my + h) % N_DEV)

            cw_rdma.wait()
            if h < CW_HOPS - 1:
                pl.semaphore_signal(
                    cw_credit, inc=1,
                    device_id=(left,), device_id_type=pl.DeviceIdType.MESH,
                )
            cmp_ref[rs, 0, :, :] = cw_ref[rs, :, :]
            if h < CCW_HOPS:
                ccw_rdma.wait()
                if h < CCW_HOPS - 1:
                    pl.semaphore_signal(
                        ccw_credit, inc=1,
                        device_id=(right,), device_id_type=pl.DeviceIdType.MESH,
                    )
                cmp_ref[rs, 1, :, :] = ccw_ref[rs, :, :]

        ts = CW_HOPS % 2
        drain((ts, 0))
        res_ref[ts, pl.ds(0, m_per), :] = jnp.dot(
            cmp_ref[ts, 0, :, :], b_bf[:, :],
            preferred_element_type=jnp.float32,
        )
        store_half(ts, 0, (my - CW_HOPS) % N_DEV)
        for key in list(stores):
            drain(key)

    return pl.pallas_call(
        body,
        out_shape=jax.ShapeDtypeStruct((out_m, n), jnp.float32),
        in_specs=[
            pl.BlockSpec(memory_space=pltpu.VMEM),
            pl.BlockSpec(memory_space=pltpu.VMEM),
        ],
        out_specs=pl.BlockSpec(memory_space=pl.ANY),
        scratch_shapes=[
            pltpu.VMEM((2, m_per, k), jnp.bfloat16),
            pltpu.VMEM((2, m_per, k), jnp.bfloat16),
            pltpu.VMEM((2, 2, m_per, k), jnp.bfloat16),
            pltpu.SemaphoreType.DMA((2,)),
            pltpu.SemaphoreType.DMA((2,)),
            pltpu.SemaphoreType.DMA((2,)),
            pltpu.SemaphoreType.DMA((2,)),
            pltpu.VMEM((2, 2 * m_per, n), jnp.float32),
            pltpu.SemaphoreType.DMA((2,)),
            pltpu.SemaphoreType.DMA((2,)),
            pltpu.VMEM((k, n), jnp.bfloat16),
            pltpu.SemaphoreType.REGULAR,
            pltpu.SemaphoreType.REGULAR,
        ],
        compiler_params=pltpu.CompilerParams(collective_id=0),
    )(A, B)


# device time: 174186 ns/iter; 1.0821x vs baseline; 1.0821x over previous
import jax
import jax.numpy as jnp
from jax import lax
from jax.experimental import pallas as pl
from jax.experimental.pallas import tpu as pltpu

N_DEV = 16
CW_HOPS = 8
CCW_HOPS = 7
N_SLOT = 4


def kernel(A, B):
    m_per, k = A.shape
    _, n = B.shape
    out_m = N_DEV * m_per

    def body(a_ref, b_ref, out_ref,
             cw_ref, ccw_ref, cmp_ref,
             cw_send, cw_recv, ccw_send, ccw_recv,
             res_ref, st0_sems, st1_sems, b_bf,
             cw_credit, ccw_credit):
        my = lax.axis_index("i")
        left = (my - 1) % N_DEV
        right = (my + 1) % N_DEV

        barrier_sem = pltpu.get_barrier_semaphore()
        for nbr in (left, right):
            pl.semaphore_signal(
                barrier_sem, inc=1,
                device_id=(nbr,), device_id_type=pl.DeviceIdType.MESH,
            )
        pl.semaphore_wait(barrier_sem, 2)

        a_bf = a_ref[:, :].astype(jnp.bfloat16)
        b_bf[:, :] = b_ref[:, :].astype(jnp.bfloat16)

        cw_ref[N_SLOT - 1, :, :] = a_bf
        ccw_ref[N_SLOT - 1, :, :] = a_bf

        pl.semaphore_signal(
            cw_credit, inc=2,
            device_id=(left,), device_id_type=pl.DeviceIdType.MESH,
        )
        pl.semaphore_signal(
            ccw_credit, inc=2,
            device_id=(right,), device_id_type=pl.DeviceIdType.MESH,
        )

        stores = {}

        def drain(key):
            if key in stores:
                stores[key].wait()
                del stores[key]

        def store_half(res_slot, half, origin):
            sems = st0_sems if half == 0 else st1_sems
            cp = pltpu.make_async_copy(
                res_ref.at[res_slot, pl.ds(half * m_per, m_per), :],
                out_ref.at[pl.ds(origin * m_per, m_per), :],
                sems.at[res_slot],
            )
            cp.start()
            stores[(res_slot, half)] = cp

        for h in range(CW_HOPS):
            src = (h - 1) % N_SLOT
            dst = h % N_SLOT
            res_slot = h % 2

            pl.semaphore_wait(cw_credit, 1)
            cw_rdma = pltpu.make_async_remote_copy(
                src_ref=cw_ref.at[src],
                dst_ref=cw_ref.at[dst],
                send_sem=cw_send.at[dst],
                recv_sem=cw_recv.at[dst],
                device_id=(right,),
                device_id_type=pl.DeviceIdType.MESH,
            )
            cw_rdma.start()
            if h < CCW_HOPS:
                pl.semaphore_wait(ccw_credit, 1)
                ccw_rdma = pltpu.make_async_remote_copy(
                    src_ref=ccw_ref.at[src],
                    dst_ref=ccw_ref.at[dst],
                    send_sem=ccw_send.at[dst],
                    recv_sem=ccw_recv.at[dst],
                    device_id=(left,),
                    device_id_type=pl.DeviceIdType.MESH,
                )
                ccw_rdma.start()

            drain((res_slot, 0))
            drain((res_slot, 1))
            if h == 0:
                res_ref[0, pl.ds(0, m_per), :] = jnp.dot(
                    a_bf, b_bf[:, :], preferred_element_type=jnp.float32
                )
                store_half(0, 0, my)
            else:
                res_ref[res_slot, :, :] = jnp.dot(
                    cmp_ref[(h - 1) % 2].reshape(2 * m_per, k), b_bf[:, :],
                    preferred_element_type=jnp.float32,
                )
                store_half(res_slot, 0, (my - h) % N_DEV)
                store_half(res_slot, 1, (my + h) % N_DEV)

            cw_rdma.wait()
            if h < CW_HOPS - 2:
                pl.semaphore_signal(
                    cw_credit, inc=1,
                    device_id=(left,), device_id_type=pl.DeviceIdType.MESH,
                )
            cmp_ref[h % 2, 0, :, :] = cw_ref[dst, :, :]
            if h < CCW_HOPS:
                ccw_rdma.wait()
                if h < CCW_HOPS - 2:
                    pl.semaphore_signal(
                        ccw_credit, inc=1,
                        device_id=(right,), device_id_type=pl.DeviceIdType.MESH,
                    )
                cmp_ref[h % 2, 1, :, :] = ccw_ref[dst, :, :]

        ts = CW_HOPS % 2
        drain((ts, 0))
        res_ref[ts, pl.ds(0, m_per), :] = jnp.dot(
            cmp_ref[(CW_HOPS - 1) % 2, 0, :, :], b_bf[:, :],
            preferred_element_type=jnp.float32,
        )
        store_half(ts, 0, (my - CW_HOPS) % N_DEV)
        for key in list(stores):
            drain(key)

    return pl.pallas_call(
        body,
        out_shape=jax.ShapeDtypeStruct((out_m, n), jnp.float32),
        in_specs=[
            pl.BlockSpec(memory_space=pltpu.VMEM),
            pl.BlockSpec(memory_space=pltpu.VMEM),
        ],
        out_specs=pl.BlockSpec(memory_space=pl.ANY),
        scratch_shapes=[
            pltpu.VMEM((N_SLOT, m_per, k), jnp.bfloat16),
            pltpu.VMEM((N_SLOT, m_per, k), jnp.bfloat16),
            pltpu.VMEM((2, 2, m_per, k), jnp.bfloat16),
            pltpu.SemaphoreType.DMA((N_SLOT,)),
            pltpu.SemaphoreType.DMA((N_SLOT,)),
            pltpu.SemaphoreType.DMA((N_SLOT,)),
            pltpu.SemaphoreType.DMA((N_SLOT,)),
            pltpu.VMEM((2, 2 * m_per, n), jnp.float32),
            pltpu.SemaphoreType.DMA((2,)),
            pltpu.SemaphoreType.DMA((2,)),
            pltpu.VMEM((k, n), jnp.bfloat16),
            pltpu.SemaphoreType.REGULAR,
            pltpu.SemaphoreType.REGULAR,
        ],
        compiler_params=pltpu.CompilerParams(collective_id=0),
    )(A, B)


# device time: 174111 ns/iter; 1.0826x vs baseline; 1.0004x over previous
import jax
import jax.numpy as jnp
from jax import lax
from jax.experimental import pallas as pl
from jax.experimental.pallas import tpu as pltpu

N_DEV = 16
CW_HOPS = 8
CCW_HOPS = 7
N_SLOT = 4


def kernel(A, B):
    m_per, k = A.shape
    _, n = B.shape
    out_m = N_DEV * m_per

    def body(a_ref, b_ref, out_ref,
             cw_ref, ccw_ref, cmp_ref,
             cw_send, cw_recv, ccw_send, ccw_recv,
             res_ref, st0_sems, st1_sems, b_bf,
             stage0_sems, stage1_sems,
             cw_credit, ccw_credit):
        my = lax.axis_index("i")
        left = (my - 1) % N_DEV
        right = (my + 1) % N_DEV

        barrier_sem = pltpu.get_barrier_semaphore()
        for nbr in (left, right):
            pl.semaphore_signal(
                barrier_sem, inc=1,
                device_id=(nbr,), device_id_type=pl.DeviceIdType.MESH,
            )
        pl.semaphore_wait(barrier_sem, 2)

        a_bf = a_ref[:, :].astype(jnp.bfloat16)
        b_bf[:, :] = b_ref[:, :].astype(jnp.bfloat16)

        cw_ref[N_SLOT - 1, :, :] = a_bf
        ccw_ref[N_SLOT - 1, :, :] = a_bf

        pl.semaphore_signal(
            cw_credit, inc=2,
            device_id=(left,), device_id_type=pl.DeviceIdType.MESH,
        )
        pl.semaphore_signal(
            ccw_credit, inc=2,
            device_id=(right,), device_id_type=pl.DeviceIdType.MESH,
        )

        stores = {}
        stages = {}

        def drain(key):
            if key in stores:
                stores[key].wait()
                del stores[key]

        def drain_stage(key):
            if key in stages:
                stages[key].wait()
                del stages[key]

        def stage(comm, cmp_slot, direction, dst_slot):
            sems = stage0_sems if direction == 0 else stage1_sems
            cp = pltpu.make_async_copy(
                comm.at[dst_slot],
                cmp_ref.at[cmp_slot, direction],
                sems.at[cmp_slot],
            )
            cp.start()
            stages[(cmp_slot, direction)] = cp

        def store_half(res_slot, half, origin):
            sems = st0_sems if half == 0 else st1_sems
            cp = pltpu.make_async_copy(
                res_ref.at[res_slot, pl.ds(half * m_per, m_per), :],
                out_ref.at[pl.ds(origin * m_per, m_per), :],
                sems.at[res_slot],
            )
            cp.start()
            stores[(res_slot, half)] = cp

        for h in range(CW_HOPS):
            src = (h - 1) % N_SLOT
            dst = h % N_SLOT
            res_slot = h % 2

            pl.semaphore_wait(cw_credit, 1)
            cw_rdma = pltpu.make_async_remote_copy(
                src_ref=cw_ref.at[src],
                dst_ref=cw_ref.at[dst],
                send_sem=cw_send.at[dst],
                recv_sem=cw_recv.at[dst],
                device_id=(right,),
                device_id_type=pl.DeviceIdType.MESH,
            )
            cw_rdma.start()
            if h < CCW_HOPS:
                pl.semaphore_wait(ccw_credit, 1)
                ccw_rdma = pltpu.make_async_remote_copy(
                    src_ref=ccw_ref.at[src],
                    dst_ref=ccw_ref.at[dst],
                    send_sem=ccw_send.at[dst],
                    recv_sem=ccw_recv.at[dst],
                    device_id=(left,),
                    device_id_type=pl.DeviceIdType.MESH,
                )
                ccw_rdma.start()

            drain((res_slot, 0))
            drain((res_slot, 1))
            if h == 0:
                res_ref[0, pl.ds(0, m_per), :] = jnp.dot(
                    a_bf, b_bf[:, :], preferred_element_type=jnp.float32
                )
                store_half(0, 0, my)
            else:
                drain_stage(((h - 1) % 2, 0))
                drain_stage(((h - 1) % 2, 1))
                res_ref[res_slot, :, :] = jnp.dot(
                    cmp_ref[(h - 1) % 2].reshape(2 * m_per, k), b_bf[:, :],
                    preferred_element_type=jnp.float32,
                )
                store_half(res_slot, 0, (my - h) % N_DEV)
                store_half(res_slot, 1, (my + h) % N_DEV)

            cw_rdma.wait()
            if h < CW_HOPS - 2:
                pl.semaphore_signal(
                    cw_credit, inc=1,
                    device_id=(left,), device_id_type=pl.DeviceIdType.MESH,
                )
            stage(cw_ref, h % 2, 0, dst)
            if h < CCW_HOPS:
                ccw_rdma.wait()
                if h < CCW_HOPS - 2:
                    pl.semaphore_signal(
                        ccw_credit, inc=1,
                        device_id=(right,), device_id_type=pl.DeviceIdType.MESH,
                    )
                stage(ccw_ref, h % 2, 1, dst)

        ts = CW_HOPS % 2
        drain((ts, 0))
        drain_stage(((CW_HOPS - 1) % 2, 0))
        res_ref[ts, pl.ds(0, m_per), :] = jnp.dot(
            cmp_ref[(CW_HOPS - 1) % 2, 0, :, :], b_bf[:, :],
            preferred_element_type=jnp.float32,
        )
        store_half(ts, 0, (my - CW_HOPS) % N_DEV)
        for key in list(stages):
            drain_stage(key)
        for key in list(stores):
            drain(key)

    return pl.pallas_call(
        body,
        out_shape=jax.ShapeDtypeStruct((out_m, n), jnp.float32),
        in_specs=[
            pl.BlockSpec(memory_space=pltpu.VMEM),
            pl.BlockSpec(memory_space=pltpu.VMEM),
        ],
        out_specs=pl.BlockSpec(memory_space=pl.ANY),
        scratch_shapes=[
            pltpu.VMEM((N_SLOT, m_per, k), jnp.bfloat16),
            pltpu.VMEM((N_SLOT, m_per, k), jnp.bfloat16),
            pltpu.VMEM((2, 2, m_per, k), jnp.bfloat16),
            pltpu.SemaphoreType.DMA((N_SLOT,)),
            pltpu.SemaphoreType.DMA((N_SLOT,)),
            pltpu.SemaphoreType.DMA((N_SLOT,)),
            pltpu.SemaphoreType.DMA((N_SLOT,)),
            pltpu.VMEM((2, 2 * m_per, n), jnp.float32),
            pltpu.SemaphoreType.DMA((2,)),
            pltpu.SemaphoreType.DMA((2,)),
            pltpu.VMEM((k, n), jnp.bfloat16),
            pltpu.SemaphoreType.DMA((2,)),
            pltpu.SemaphoreType.DMA((2,)),
            pltpu.SemaphoreType.REGULAR,
            pltpu.SemaphoreType.REGULAR,
        ],
        compiler_params=pltpu.CompilerParams(collective_id=0),
    )(A, B)
